# baseline (speedup 1.0000x reference)
import functools

import jax
import jax.numpy as jnp
from jax import lax
from jax.experimental import pallas as pl
from jax.experimental.pallas import tpu as pltpu
from jax.experimental.pallas import tpu_sc as plsc

BATCH = 16384
FACTORS = 32
PACK = 128 // FACTORS

_INFO = plsc.get_sparse_core_info()
_NC = _INFO.num_cores
_NS = _INFO.num_subcores
_NW = _NC * _NS
_BPW = BATCH // _NW
_CHUNK = 128
_NCHUNK = _BPW // _CHUNK
_L = _INFO.num_lanes


def _sc_gather_packed(user_idx, item_idx, uemb128, iemb128):
    mesh = plsc.VectorSubcoreMesh(core_axis_name="c", subcore_axis_name="s")

    @functools.partial(
        pl.kernel,
        mesh=mesh,
        out_type=[
            jax.ShapeDtypeStruct((BATCH, 128), jnp.float32),
            jax.ShapeDtypeStruct((BATCH, 128), jnp.float32),
        ],
        scratch_types=[
            pltpu.VMEM((_BPW,), jnp.int32),
            pltpu.VMEM((_BPW,), jnp.int32),
            pltpu.VMEM((_CHUNK, 128), jnp.float32),
            pltpu.VMEM((_CHUNK, 128), jnp.float32),
            pltpu.VMEM((_CHUNK, 128), jnp.float32),
            pltpu.VMEM((_CHUNK, 128), jnp.float32),
            pltpu.SemaphoreType.DMA,
            pltpu.SemaphoreType.DMA,
        ],
    )
    def k(uidx_hbm, iidx_hbm, uemb_hbm, iemb_hbm, u_out, v_out,
          uj_v, ij_v, ub0, ub1, ib0, ib1, gsem, wsem):
        wid = lax.axis_index("s") * _NC + lax.axis_index("c")
        base = wid * _BPW
        pltpu.sync_copy(uidx_hbm.at[pl.ds(base, _BPW)], uj_v)
        pltpu.sync_copy(iidx_hbm.at[pl.ds(base, _BPW)], ij_v)
        for i in range(_BPW // _L):
            sl = pl.ds(i * _L, _L)
            uj_v[sl] = lax.shift_right_logical(uj_v[sl], 2)
            ij_v[sl] = lax.shift_right_logical(ij_v[sl], 2)
        ubufs, ibufs = (ub0, ub1), (ib0, ib1)
        uwb = [None, None]
        iwb = [None, None]
        for c in range(_NCHUNK):
            sl = pl.ds(c * _CHUNK, _CHUNK)
            b = c % 2
            if uwb[b] is not None:
                uwb[b].wait()
                iwb[b].wait()
            gu = pltpu.async_copy(uemb_hbm.at[uj_v.at[sl]], ubufs[b], gsem)
            gi = pltpu.async_copy(iemb_hbm.at[ij_v.at[sl]], ibufs[b], gsem)
            gu.wait()
            gi.wait()
            osl = pl.ds(base + c * _CHUNK, _CHUNK)
            uwb[b] = pltpu.async_copy(ubufs[b], u_out.at[osl], wsem)
            iwb[b] = pltpu.async_copy(ibufs[b], v_out.at[osl], wsem)
        for b in range(2):
            if uwb[b] is not None:
                uwb[b].wait()
                iwb[b].wait()

    return k(user_idx, item_idx, uemb128, iemb128)


_BM = 2048
_G = BATCH // _BM


def _select_group(x128, sel):
    out = jnp.zeros((x128.shape[0], FACTORS), jnp.float32)
    for kk in range(PACK):
        out = out + jnp.where(sel == kk,
                              x128[:, kk * FACTORS:(kk + 1) * FACTORS], 0.0)
    return out


def _mlp_body(u_ref, v_ref, usel_ref, isel_ref, w1_ref, b1_ref, w2_ref,
              b2_ref, w3_ref, b3_ref, w4_ref, b4_ref, o_ref):
    f32 = jnp.float32
    usel = usel_ref[0] & (PACK - 1)
    isel = isel_ref[0] & (PACK - 1)
    u = _select_group(u_ref[...], usel)
    v = _select_group(v_ref[...], isel)
    w1 = w1_ref[...]
    h = (jnp.dot(u, w1[:FACTORS], preferred_element_type=f32)
         + jnp.dot(v, w1[FACTORS:], preferred_element_type=f32)
         + b1_ref[...])
    h = jnp.maximum(h, 0.0)
    h = jnp.dot(h, w2_ref[...], preferred_element_type=f32) + b2_ref[...]
    h = jnp.maximum(h, 0.0)
    h = jnp.dot(h, w3_ref[...], preferred_element_type=f32) + b3_ref[...]
    h = jnp.maximum(h, 0.0)
    s = jnp.sum(h * w4_ref[...], axis=1, keepdims=True) + b4_ref[...]
    o_ref[...] = jax.nn.sigmoid(s)


def _mlp(u128, v128, uidx3, iidx3, W1, b1, W2, b2, W3, b3, W4, b4):
    out = pl.pallas_call(
        _mlp_body,
        grid=(_G,),
        in_specs=[
            pl.BlockSpec((_BM, 128), lambda i: (i, 0)),
            pl.BlockSpec((_BM, 128), lambda i: (i, 0)),
            pl.BlockSpec((1, _BM, 1), lambda i: (i, 0, 0)),
            pl.BlockSpec((1, _BM, 1), lambda i: (i, 0, 0)),
            pl.BlockSpec((64, 64), lambda i: (0, 0)),
            pl.BlockSpec((1, 64), lambda i: (0, 0)),
            pl.BlockSpec((64, 32), lambda i: (0, 0)),
            pl.BlockSpec((1, 32), lambda i: (0, 0)),
            pl.BlockSpec((32, 16), lambda i: (0, 0)),
            pl.BlockSpec((1, 16), lambda i: (0, 0)),
            pl.BlockSpec((1, 16), lambda i: (0, 0)),
            pl.BlockSpec((1, 1), lambda i: (0, 0)),
        ],
        out_specs=pl.BlockSpec((_BM, 1), lambda i: (i, 0)),
        out_shape=jax.ShapeDtypeStruct((BATCH, 1), jnp.float32),
    )(u128, v128, uidx3, iidx3, W1, b1.reshape(1, 64), W2, b2.reshape(1, 32),
      W3, b3.reshape(1, 16), W4.reshape(1, 16), b4.reshape(1, 1))
    return jnp.squeeze(out, axis=-1)


def kernel(user_input, item_input, user_emb, item_emb,
           W1, b1, W2, b2, W3, b3, W4, b4):
    uemb128 = user_emb.reshape(-1, 128)
    iemb128 = item_emb.reshape(-1, 128)
    u128, v128 = _sc_gather_packed(user_input, item_input, uemb128, iemb128)
    uidx3 = user_input.reshape(_G, _BM, 1)
    iidx3 = item_input.reshape(_G, _BM, 1)
    return _mlp(u128, v128, uidx3, iidx3, W1, b1, W2, b2, W3, b3, W4, b4)
